# Initial kernel scaffold; baseline (speedup 1.0000x reference)
import jax
import jax.numpy as jnp
from jax.experimental import pallas as pl


def kernel(x, y, W_ls, b_ls, W_key, b_key, W_q, b_q, W_p1, b_p1, g_p, be_p, W_p2, b_p2, W_a1, b_a1, g_a, be_a, W_a2, b_a2, W_le, b_le):
    raise NotImplementedError("write your pallas kernel here")


# speedup vs baseline: 14.6011x; 14.6011x over previous
import functools

import jax
import jax.numpy as jnp
from jax import lax
from jax.experimental import pallas as pl
from jax.experimental.pallas import tpu as pltpu
from jax.experimental.pallas import tpu_sc as plsc

D = 64
K = 16
B = 4
N = 4096
BN = B * N
BNK = BN * K
EPS = 1e-5

TN = 256
GR = 256
NW = 32
CHUNK = 128
PW = BNK // NW


def _proj_knn_body(xall_ref, xt_ref, yt_ref, wls_ref, wkey_ref, wq_ref,
                   bias_ref, y1_ref, tab_ref, q_ref, idx_ref):
    b = pl.program_id(0)
    xall = xall_ref[0]
    xt = xt_ref[...]
    yt = yt_ref[...]
    y1 = jnp.dot(yt, wls_ref[...], preferred_element_type=jnp.float32)
    y1 = y1 + bias_ref[0:1, :]
    key = jnp.dot(y1, wkey_ref[...], preferred_element_type=jnp.float32)
    key = key + bias_ref[1:2, :]
    q = jnp.dot(xt, wq_ref[...], preferred_element_type=jnp.float32)
    q = q + bias_ref[2:3, :]
    y1_ref[...] = y1
    tab_ref[...] = jnp.concatenate(
        [key, jnp.pad(xt, ((0, 0), (0, 56)))], axis=1)
    q_ref[...] = q

    sqa = jnp.sum(xall * xall, axis=0, keepdims=True)
    sqt = jnp.sum(xt * xt, axis=1, keepdims=True)
    d2 = sqt + sqa - 2.0 * jnp.dot(xt, xall, preferred_element_type=jnp.float32)
    iota = lax.broadcasted_iota(jnp.int32, (TN, N), 1)
    lane = lax.broadcasted_iota(jnp.int32, (TN, K), 1)
    acc = jnp.zeros((TN, K), jnp.int32)
    d = d2
    for k in range(K):
        m = jnp.min(d, axis=1, keepdims=True)
        ik = jnp.min(jnp.where(d == m, iota, N), axis=1, keepdims=True)
        acc = jnp.where(lane == k, ik, acc)
        d = jnp.where(iota == ik, jnp.inf, d)
    idx_ref[...] = acc + b * N


def _proj_knn(xp, xrp, yT, wlsT, wkeyT, wqT, bias):
    nt = N // TN
    f32 = jnp.float32
    return pl.pallas_call(
        _proj_knn_body,
        grid=(B, nt),
        in_specs=[
            pl.BlockSpec((1, 8, N), lambda b, i: (b, 0, 0)),
            pl.BlockSpec((TN, 8), lambda b, i: (b * nt + i, 0)),
            pl.BlockSpec((TN, D), lambda b, i: (b * nt + i, 0)),
            pl.BlockSpec((D, D), lambda b, i: (0, 0)),
            pl.BlockSpec((D, D), lambda b, i: (0, 0)),
            pl.BlockSpec((8, D), lambda b, i: (0, 0)),
            pl.BlockSpec((8, D), lambda b, i: (0, 0)),
        ],
        out_specs=[
            pl.BlockSpec((TN, D), lambda b, i: (b * nt + i, 0)),
            pl.BlockSpec((TN, 2 * D), lambda b, i: (b * nt + i, 0)),
            pl.BlockSpec((TN, D), lambda b, i: (b * nt + i, 0)),
            pl.BlockSpec((TN, K), lambda b, i: (b * nt + i, 0)),
        ],
        out_shape=[
            jax.ShapeDtypeStruct((BN, D), f32),
            jax.ShapeDtypeStruct((BN, 2 * D), f32),
            jax.ShapeDtypeStruct((BN, D), f32),
            jax.ShapeDtypeStruct((BN, K), jnp.int32),
        ],
    )(xp, xrp, yT, wlsT, wkeyT, wqT, bias)


def _sc_gather_body(tab_hbm, idx_hbm, g_hbm, idx_v, buf, sem):
    wid = lax.axis_index("s") * 2 + lax.axis_index("c")
    base = wid * PW
    pltpu.sync_copy(idx_hbm.at[pl.ds(base, PW)], idx_v)

    def body(i, carry):
        ids = idx_v.at[pl.ds(i * CHUNK, CHUNK)]
        pltpu.async_copy(tab_hbm.at[ids], buf, sem).wait()
        pltpu.sync_copy(buf, g_hbm.at[pl.ds(base + i * CHUNK, CHUNK)])
        return carry

    lax.fori_loop(0, PW // CHUNK, body, 0)


def _sc_gather(tab_r, idx_flat):
    f32 = jnp.float32
    mesh = plsc.VectorSubcoreMesh(core_axis_name="c", subcore_axis_name="s")
    kern = pl.kernel(
        _sc_gather_body,
        mesh=mesh,
        out_type=jax.ShapeDtypeStruct((BNK, 2 * D), f32),
        scratch_types=[
            pltpu.VMEM((PW,), jnp.int32),
            pltpu.VMEM((CHUNK, 2 * D), f32),
            pltpu.SemaphoreType.DMA,
        ],
    )
    return kern(tab_r, idx_flat)


def _accum(ref, part):
    @pl.when(pl.program_id(0) == 0)
    def _():
        ref[...] = part

    @pl.when(pl.program_id(0) != 0)
    def _():
        ref[...] += part


def _pos_stats_body(g_ref, xr_ref, sp_ref, mp_ref):
    xg = g_ref[...][:, :, D:D + 16]
    pr = xr_ref[...][:, None, :] - xg
    pr2 = pr.reshape(pr.shape[0] * K, 16)
    m = lax.dot_general(pr2, pr2, (((0,), (0,)), ((), ())),
                        preferred_element_type=jnp.float32)
    s = jnp.sum(pr2, axis=0, keepdims=True)
    _accum(mp_ref, m)
    _accum(sp_ref, jnp.broadcast_to(s, (8, 16)))


def _pos_stats(g3, x16):
    gp = 512
    nt = BN // gp
    f32 = jnp.float32
    return pl.pallas_call(
        _pos_stats_body,
        grid=(nt,),
        in_specs=[
            pl.BlockSpec((gp, K, 2 * D), lambda i: (i, 0, 0)),
            pl.BlockSpec((gp, 16), lambda i: (i, 0)),
        ],
        out_specs=[
            pl.BlockSpec((8, 16), lambda i: (0, 0)),
            pl.BlockSpec((16, 16), lambda i: (0, 0)),
        ],
        out_shape=[
            jax.ShapeDtypeStruct((8, 16), f32),
            jax.ShapeDtypeStruct((16, 16), f32),
        ],
    )(g3, x16)


def _pos_branch(xg, xr_ref, w1f_ref, wp2_ref, pk_ref):
    gr = xr_ref.shape[0]
    pr = xr_ref[...][:, None, :] - xg
    pr2 = pr.reshape(gr * K, 16)
    h = jnp.dot(pr2, w1f_ref[...], preferred_element_type=jnp.float32)
    h = jnp.maximum(h + pk_ref[0:1, :], 0.0)
    pe = jnp.dot(h, wp2_ref[...], preferred_element_type=jnp.float32)
    return pe + pk_ref[1:2, :]


def _t_stats_body(g_ref, xr_ref, q_ref, w1f_ref, wp2_ref, pk_ref,
                  st_ref, mt_ref):
    g = g_ref[...]
    pe = _pos_branch(g[:, :, D:D + 16], xr_ref, w1f_ref, wp2_ref, pk_ref)
    qk = q_ref[...][:, None, :] - g[:, :, :D]
    t = qk.reshape(GR * K, D) + pe
    m = lax.dot_general(t, t, (((0,), (0,)), ((), ())),
                        preferred_element_type=jnp.float32)
    s = jnp.sum(t, axis=0, keepdims=True)
    _accum(mt_ref, m)
    _accum(st_ref, jnp.broadcast_to(s, (8, D)))


def _t_stats(g3, x16, q_r, w1f, wp2T, pk64):
    nt = BN // GR
    f32 = jnp.float32
    return pl.pallas_call(
        _t_stats_body,
        grid=(nt,),
        in_specs=[
            pl.BlockSpec((GR, K, 2 * D), lambda i: (i, 0, 0)),
            pl.BlockSpec((GR, 16), lambda i: (i, 0)),
            pl.BlockSpec((GR, D), lambda i: (i, 0)),
            pl.BlockSpec((16, D), lambda i: (0, 0)),
            pl.BlockSpec((D, D), lambda i: (0, 0)),
            pl.BlockSpec((8, D), lambda i: (0, 0)),
        ],
        out_specs=[
            pl.BlockSpec((8, D), lambda i: (0, 0)),
            pl.BlockSpec((D, D), lambda i: (0, 0)),
        ],
        out_shape=[
            jax.ShapeDtypeStruct((8, D), f32),
            jax.ShapeDtypeStruct((D, D), f32),
        ],
    )(g3, x16, q_r, w1f, wp2T, pk64)


def _final_body(g_ref, xr_ref, q_ref, v_ref, id_ref,
                w1f_ref, wp2_ref, pk_ref, a1f_ref, pk256_ref, wa2_ref,
                wle_ref, out_ref):
    g = g_ref[...]
    pe = _pos_branch(g[:, :, D:D + 16], xr_ref, w1f_ref, wp2_ref, pk_ref)
    qk = q_ref[...][:, None, :] - g[:, :, :D]
    t = qk.reshape(GR * K, D) + pe
    a = jnp.dot(t, a1f_ref[...], preferred_element_type=jnp.float32)
    a = jnp.maximum(a + pk256_ref[0:1, :], 0.0)
    lg = jnp.dot(a, wa2_ref[...], preferred_element_type=jnp.float32)
    lg = lg + pk_ref[2:3, :]
    lg3 = lg.reshape(GR, K, D)
    mx = jnp.max(lg3, axis=1, keepdims=True)
    e = jnp.exp(lg3 - mx)
    attn = e / jnp.sum(e, axis=1, keepdims=True)
    val = v_ref[...][:, None, :] + pe.reshape(GR, K, D)
    agg = jnp.sum(attn * val, axis=1)
    out = jnp.dot(agg, wle_ref[...], preferred_element_type=jnp.float32)
    out_ref[...] = out + pk_ref[3:4, :] + id_ref[...]


def _final(g3, x16, q_r, y1_r, yT, w1f, wp2T, pk64, a1f, pk256, wa2T, wleT):
    nt = BN // GR
    f32 = jnp.float32
    return pl.pallas_call(
        _final_body,
        grid=(nt,),
        in_specs=[
            pl.BlockSpec((GR, K, 2 * D), lambda i: (i, 0, 0)),
            pl.BlockSpec((GR, 16), lambda i: (i, 0)),
            pl.BlockSpec((GR, D), lambda i: (i, 0)),
            pl.BlockSpec((GR, D), lambda i: (i, 0)),
            pl.BlockSpec((GR, D), lambda i: (i, 0)),
            pl.BlockSpec((16, D), lambda i: (0, 0)),
            pl.BlockSpec((D, D), lambda i: (0, 0)),
            pl.BlockSpec((8, D), lambda i: (0, 0)),
            pl.BlockSpec((D, 4 * D), lambda i: (0, 0)),
            pl.BlockSpec((8, 4 * D), lambda i: (0, 0)),
            pl.BlockSpec((4 * D, D), lambda i: (0, 0)),
            pl.BlockSpec((D, D), lambda i: (0, 0)),
        ],
        out_specs=[pl.BlockSpec((GR, D), lambda i: (i, 0))],
        out_shape=[jax.ShapeDtypeStruct((BN, D), f32)],
    )(g3, x16, q_r, y1_r, yT, w1f, wp2T, pk64, a1f, pk256, wa2T, wleT)[0]


def kernel(x, y, W_ls, b_ls, W_key, b_key, W_q, b_q, W_p1, b_p1, g_p, be_p,
           W_p2, b_p2, W_a1, b_a1, g_a, be_a, W_a2, b_a2, W_le, b_le):
    f32 = jnp.float32
    xp = jnp.pad(x, ((0, 0), (0, 5), (0, 0)))
    xr = jnp.transpose(x, (0, 2, 1))
    xrp = jnp.pad(xr, ((0, 0), (0, 0), (0, 5))).reshape(BN, 8)
    x16 = jnp.pad(xr, ((0, 0), (0, 0), (0, 13))).reshape(BN, 16)
    yT = jnp.transpose(y, (0, 2, 1)).reshape(BN, D)

    W_qp = jnp.pad(W_q, ((0, 0), (0, 5)))
    bias = jnp.zeros((8, D), f32).at[0].set(b_ls).at[1].set(b_key).at[2].set(b_q)

    y1_r, tab_r, q_r, idxg = _proj_knn(
        xp, xrp, yT, W_ls.T, W_key.T, W_qp.T, bias)

    g3 = _sc_gather(tab_r, idxg.reshape(BNK)).reshape(BN, K, 2 * D)

    sp, mp = _pos_stats(g3, x16)
    cnt = float(BNK)
    mu_p = sp[0] / cnt
    cov_p = mp / cnt - jnp.outer(mu_p, mu_p)
    W1p = jnp.pad(W_p1, ((0, 0), (0, 13)))
    m_o = W1p @ mu_p + b_p1
    v_o = ((W1p @ cov_p) * W1p).sum(1)
    s_p = g_p / jnp.sqrt(v_o + EPS)
    c_p = s_p * (b_p1 - m_o) + be_p
    w1f = W1p.T * s_p[None, :]
    pk64 = jnp.zeros((8, D), f32).at[0].set(c_p).at[1].set(b_p2)

    st, mt = _t_stats(g3, x16, q_r, w1f, W_p2.T, pk64)
    mu_t = st[0] / cnt
    cov_t = mt / cnt - jnp.outer(mu_t, mu_t)
    m_a = W_a1 @ mu_t + b_a1
    v_a = ((W_a1 @ cov_t) * W_a1).sum(1)
    s_a = g_a / jnp.sqrt(v_a + EPS)
    c_a = s_a * (b_a1 - m_a) + be_a
    a1f = W_a1.T * s_a[None, :]
    pk64 = pk64.at[2].set(b_a2).at[3].set(b_le)
    pk256 = jnp.zeros((8, 4 * D), f32).at[0].set(c_a)

    out_r = _final(g3, x16, q_r, y1_r, yT,
                   w1f, W_p2.T, pk64, a1f, pk256, W_a2.T, W_le.T)
    return jnp.transpose(out_r.reshape(B, N, D), (0, 2, 1))


# speedup vs baseline: 16.0121x; 1.0966x over previous
import functools

import jax
import jax.numpy as jnp
from jax import lax
from jax.experimental import pallas as pl
from jax.experimental.pallas import tpu as pltpu
from jax.experimental.pallas import tpu_sc as plsc

D = 64
K = 16
B = 4
N = 4096
BN = B * N
BNK = BN * K
EPS = 1e-5

TN = 256
GR = 256
NW = 32
CHUNK = 128
PW = BNK // NW


def _proj_knn_body(xall_ref, xt_ref, yt_ref, wls_ref, wkey_ref, wq_ref,
                   bias_ref, y1_ref, tab_ref, q_ref, idx_ref):
    b = pl.program_id(0)
    xall = xall_ref[0]
    xt = xt_ref[...]
    yt = yt_ref[...]
    y1 = jnp.dot(yt, wls_ref[...], preferred_element_type=jnp.float32)
    y1 = y1 + bias_ref[0:1, :]
    key = jnp.dot(y1, wkey_ref[...], preferred_element_type=jnp.float32)
    key = key + bias_ref[1:2, :]
    q = jnp.dot(xt, wq_ref[...], preferred_element_type=jnp.float32)
    q = q + bias_ref[2:3, :]
    y1_ref[...] = y1
    tab_ref[...] = jnp.concatenate(
        [key, jnp.pad(xt, ((0, 0), (0, 56)))], axis=1)
    q_ref[...] = q

    sqa = jnp.sum(xall * xall, axis=0, keepdims=True)
    sqt = jnp.sum(xt * xt, axis=1, keepdims=True)
    d2 = sqt + sqa - 2.0 * jnp.dot(xt, xall, preferred_element_type=jnp.float32)
    iota = lax.broadcasted_iota(jnp.int32, (TN, N), 1)
    lane = lax.broadcasted_iota(jnp.int32, (TN, K), 1)
    acc = jnp.zeros((TN, K), jnp.int32)
    d = d2
    for k in range(K):
        m = jnp.min(d, axis=1, keepdims=True)
        eq = d == m
        ik = jnp.min(jnp.where(eq, iota, N), axis=1, keepdims=True)
        acc = jnp.where(lane == k, ik, acc)
        d = jnp.where(eq, jnp.inf, d)
    idx_ref[...] = acc + b * N


def _proj_knn(xp, xrp, yT, wlsT, wkeyT, wqT, bias):
    nt = N // TN
    f32 = jnp.float32
    return pl.pallas_call(
        _proj_knn_body,
        grid=(B, nt),
        in_specs=[
            pl.BlockSpec((1, 8, N), lambda b, i: (b, 0, 0)),
            pl.BlockSpec((TN, 8), lambda b, i: (b * nt + i, 0)),
            pl.BlockSpec((TN, D), lambda b, i: (b * nt + i, 0)),
            pl.BlockSpec((D, D), lambda b, i: (0, 0)),
            pl.BlockSpec((D, D), lambda b, i: (0, 0)),
            pl.BlockSpec((8, D), lambda b, i: (0, 0)),
            pl.BlockSpec((8, D), lambda b, i: (0, 0)),
        ],
        out_specs=[
            pl.BlockSpec((TN, D), lambda b, i: (b * nt + i, 0)),
            pl.BlockSpec((TN, 2 * D), lambda b, i: (b * nt + i, 0)),
            pl.BlockSpec((TN, D), lambda b, i: (b * nt + i, 0)),
            pl.BlockSpec((TN, K), lambda b, i: (b * nt + i, 0)),
        ],
        out_shape=[
            jax.ShapeDtypeStruct((BN, D), f32),
            jax.ShapeDtypeStruct((BN, 2 * D), f32),
            jax.ShapeDtypeStruct((BN, D), f32),
            jax.ShapeDtypeStruct((BN, K), jnp.int32),
        ],
    )(xp, xrp, yT, wlsT, wkeyT, wqT, bias)


def _sc_gather_body(tab_hbm, idx_hbm, g_hbm, idx_v, buf, sem):
    wid = lax.axis_index("s") * 2 + lax.axis_index("c")
    base = wid * PW
    pltpu.sync_copy(idx_hbm.at[pl.ds(base, PW)], idx_v)

    def body(i, carry):
        ids = idx_v.at[pl.ds(i * CHUNK, CHUNK)]
        pltpu.async_copy(tab_hbm.at[ids], buf, sem).wait()
        pltpu.sync_copy(buf, g_hbm.at[pl.ds(base + i * CHUNK, CHUNK)])
        return carry

    lax.fori_loop(0, PW // CHUNK, body, 0)


def _sc_gather(tab_r, idx_flat):
    f32 = jnp.float32
    mesh = plsc.VectorSubcoreMesh(core_axis_name="c", subcore_axis_name="s")
    kern = pl.kernel(
        _sc_gather_body,
        mesh=mesh,
        out_type=jax.ShapeDtypeStruct((BNK, 2 * D), f32),
        scratch_types=[
            pltpu.VMEM((PW,), jnp.int32),
            pltpu.VMEM((CHUNK, 2 * D), f32),
            pltpu.SemaphoreType.DMA,
        ],
    )
    return kern(tab_r, idx_flat)


def _accum(ref, part):
    @pl.when(pl.program_id(0) == 0)
    def _():
        ref[...] = part

    @pl.when(pl.program_id(0) != 0)
    def _():
        ref[...] += part


def _pos_stats_body(g_ref, xr_ref, sp_ref, mp_ref):
    xg = g_ref[...][:, :, D:D + 16]
    pr = xr_ref[...][:, None, :] - xg
    pr2 = pr.reshape(pr.shape[0] * K, 16)
    m = lax.dot_general(pr2, pr2, (((0,), (0,)), ((), ())),
                        preferred_element_type=jnp.float32)
    s = jnp.sum(pr2, axis=0, keepdims=True)
    _accum(mp_ref, m)
    _accum(sp_ref, jnp.broadcast_to(s, (8, 16)))


def _pos_stats(g3, x16):
    gp = 512
    nt = BN // gp
    f32 = jnp.float32
    return pl.pallas_call(
        _pos_stats_body,
        grid=(nt,),
        in_specs=[
            pl.BlockSpec((gp, K, 2 * D), lambda i: (i, 0, 0)),
            pl.BlockSpec((gp, 16), lambda i: (i, 0)),
        ],
        out_specs=[
            pl.BlockSpec((8, 16), lambda i: (0, 0)),
            pl.BlockSpec((16, 16), lambda i: (0, 0)),
        ],
        out_shape=[
            jax.ShapeDtypeStruct((8, 16), f32),
            jax.ShapeDtypeStruct((16, 16), f32),
        ],
    )(g3, x16)


def _pos_branch(xg, xr_ref, w1f_ref, wp2_ref, pk_ref):
    gr = xr_ref.shape[0]
    pr = xr_ref[...][:, None, :] - xg
    pr2 = pr.reshape(gr * K, 16)
    h = jnp.dot(pr2, w1f_ref[...], preferred_element_type=jnp.float32)
    h = jnp.maximum(h + pk_ref[0:1, :], 0.0)
    pe = jnp.dot(h, wp2_ref[...], preferred_element_type=jnp.float32)
    return pe + pk_ref[1:2, :]


def _t_stats_body(g_ref, xr_ref, q_ref, w1f_ref, wp2_ref, pk_ref,
                  st_ref, mt_ref):
    g = g_ref[...]
    pe = _pos_branch(g[:, :, D:D + 16], xr_ref, w1f_ref, wp2_ref, pk_ref)
    qk = q_ref[...][:, None, :] - g[:, :, :D]
    t = qk.reshape(GR * K, D) + pe
    m = lax.dot_general(t, t, (((0,), (0,)), ((), ())),
                        preferred_element_type=jnp.float32)
    s = jnp.sum(t, axis=0, keepdims=True)
    _accum(mt_ref, m)
    _accum(st_ref, jnp.broadcast_to(s, (8, D)))


def _t_stats(g3, x16, q_r, w1f, wp2T, pk64):
    nt = BN // GR
    f32 = jnp.float32
    return pl.pallas_call(
        _t_stats_body,
        grid=(nt,),
        in_specs=[
            pl.BlockSpec((GR, K, 2 * D), lambda i: (i, 0, 0)),
            pl.BlockSpec((GR, 16), lambda i: (i, 0)),
            pl.BlockSpec((GR, D), lambda i: (i, 0)),
            pl.BlockSpec((16, D), lambda i: (0, 0)),
            pl.BlockSpec((D, D), lambda i: (0, 0)),
            pl.BlockSpec((8, D), lambda i: (0, 0)),
        ],
        out_specs=[
            pl.BlockSpec((8, D), lambda i: (0, 0)),
            pl.BlockSpec((D, D), lambda i: (0, 0)),
        ],
        out_shape=[
            jax.ShapeDtypeStruct((8, D), f32),
            jax.ShapeDtypeStruct((D, D), f32),
        ],
    )(g3, x16, q_r, w1f, wp2T, pk64)


def _final_body(g_ref, xr_ref, q_ref, v_ref, id_ref,
                w1f_ref, wp2_ref, pk_ref, a1f_ref, pk256_ref, wa2_ref,
                wle_ref, out_ref):
    g = g_ref[...]
    pe = _pos_branch(g[:, :, D:D + 16], xr_ref, w1f_ref, wp2_ref, pk_ref)
    qk = q_ref[...][:, None, :] - g[:, :, :D]
    t = qk.reshape(GR * K, D) + pe
    a = jnp.dot(t, a1f_ref[...], preferred_element_type=jnp.float32)
    a = jnp.maximum(a + pk256_ref[0:1, :], 0.0)
    lg = jnp.dot(a, wa2_ref[...], preferred_element_type=jnp.float32)
    lg = lg + pk_ref[2:3, :]
    lg3 = lg.reshape(GR, K, D)
    mx = jnp.max(lg3, axis=1, keepdims=True)
    e = jnp.exp(lg3 - mx)
    attn = e / jnp.sum(e, axis=1, keepdims=True)
    val = v_ref[...][:, None, :] + pe.reshape(GR, K, D)
    agg = jnp.sum(attn * val, axis=1)
    out = jnp.dot(agg, wle_ref[...], preferred_element_type=jnp.float32)
    out_ref[...] = out + pk_ref[3:4, :] + id_ref[...]


def _final(g3, x16, q_r, y1_r, yT, w1f, wp2T, pk64, a1f, pk256, wa2T, wleT):
    nt = BN // GR
    f32 = jnp.float32
    return pl.pallas_call(
        _final_body,
        grid=(nt,),
        in_specs=[
            pl.BlockSpec((GR, K, 2 * D), lambda i: (i, 0, 0)),
            pl.BlockSpec((GR, 16), lambda i: (i, 0)),
            pl.BlockSpec((GR, D), lambda i: (i, 0)),
            pl.BlockSpec((GR, D), lambda i: (i, 0)),
            pl.BlockSpec((GR, D), lambda i: (i, 0)),
            pl.BlockSpec((16, D), lambda i: (0, 0)),
            pl.BlockSpec((D, D), lambda i: (0, 0)),
            pl.BlockSpec((8, D), lambda i: (0, 0)),
            pl.BlockSpec((D, 4 * D), lambda i: (0, 0)),
            pl.BlockSpec((8, 4 * D), lambda i: (0, 0)),
            pl.BlockSpec((4 * D, D), lambda i: (0, 0)),
            pl.BlockSpec((D, D), lambda i: (0, 0)),
        ],
        out_specs=[pl.BlockSpec((GR, D), lambda i: (i, 0))],
        out_shape=[jax.ShapeDtypeStruct((BN, D), f32)],
    )(g3, x16, q_r, y1_r, yT, w1f, wp2T, pk64, a1f, pk256, wa2T, wleT)[0]


def kernel(x, y, W_ls, b_ls, W_key, b_key, W_q, b_q, W_p1, b_p1, g_p, be_p,
           W_p2, b_p2, W_a1, b_a1, g_a, be_a, W_a2, b_a2, W_le, b_le):
    f32 = jnp.float32
    xp = jnp.pad(x, ((0, 0), (0, 5), (0, 0)))
    xr = jnp.transpose(x, (0, 2, 1))
    xrp = jnp.pad(xr, ((0, 0), (0, 0), (0, 5))).reshape(BN, 8)
    x16 = jnp.pad(xr, ((0, 0), (0, 0), (0, 13))).reshape(BN, 16)
    yT = jnp.transpose(y, (0, 2, 1)).reshape(BN, D)

    W_qp = jnp.pad(W_q, ((0, 0), (0, 5)))
    bias = jnp.zeros((8, D), f32).at[0].set(b_ls).at[1].set(b_key).at[2].set(b_q)

    y1_r, tab_r, q_r, idxg = _proj_knn(
        xp, xrp, yT, W_ls.T, W_key.T, W_qp.T, bias)

    g3 = _sc_gather(tab_r, idxg.reshape(BNK)).reshape(BN, K, 2 * D)

    sp, mp = _pos_stats(g3, x16)
    cnt = float(BNK)
    mu_p = sp[0] / cnt
    cov_p = mp / cnt - jnp.outer(mu_p, mu_p)
    W1p = jnp.pad(W_p1, ((0, 0), (0, 13)))
    m_o = W1p @ mu_p + b_p1
    v_o = ((W1p @ cov_p) * W1p).sum(1)
    s_p = g_p / jnp.sqrt(v_o + EPS)
    c_p = s_p * (b_p1 - m_o) + be_p
    w1f = W1p.T * s_p[None, :]
    pk64 = jnp.zeros((8, D), f32).at[0].set(c_p).at[1].set(b_p2)

    st, mt = _t_stats(g3, x16, q_r, w1f, W_p2.T, pk64)
    mu_t = st[0] / cnt
    cov_t = mt / cnt - jnp.outer(mu_t, mu_t)
    m_a = W_a1 @ mu_t + b_a1
    v_a = ((W_a1 @ cov_t) * W_a1).sum(1)
    s_a = g_a / jnp.sqrt(v_a + EPS)
    c_a = s_a * (b_a1 - m_a) + be_a
    a1f = W_a1.T * s_a[None, :]
    pk64 = pk64.at[2].set(b_a2).at[3].set(b_le)
    pk256 = jnp.zeros((8, 4 * D), f32).at[0].set(c_a)

    out_r = _final(g3, x16, q_r, y1_r, yT,
                   w1f, W_p2.T, pk64, a1f, pk256, W_a2.T, W_le.T)
    return jnp.transpose(out_r.reshape(B, N, D), (0, 2, 1))


# speedup vs baseline: 16.5174x; 1.0316x over previous
import functools

import jax
import jax.numpy as jnp
from jax import lax
from jax.experimental import pallas as pl
from jax.experimental.pallas import tpu as pltpu
from jax.experimental.pallas import tpu_sc as plsc

D = 64
K = 16
B = 4
N = 4096
BN = B * N
BNK = BN * K
EPS = 1e-5

TN = 256
GR = 256
NW = 32
CHUNK = 128
PW = BNK // NW


def _proj_knn_body(xall_ref, xt_ref, yt_ref, wls_ref, wkey_ref, wq_ref,
                   bias_ref, y1_ref, tab_ref, q_ref, idx_ref):
    b = pl.program_id(0)
    xall = xall_ref[0]
    xt = xt_ref[...]
    yt = yt_ref[...]
    y1 = jnp.dot(yt, wls_ref[...], preferred_element_type=jnp.float32)
    y1 = y1 + bias_ref[0:1, :]
    key = jnp.dot(y1, wkey_ref[...], preferred_element_type=jnp.float32)
    key = key + bias_ref[1:2, :]
    q = jnp.dot(xt, wq_ref[...], preferred_element_type=jnp.float32)
    q = q + bias_ref[2:3, :]
    y1_ref[...] = y1
    tab_ref[...] = jnp.concatenate(
        [key, jnp.pad(xt, ((0, 0), (0, 56)))], axis=1)
    q_ref[...] = q

    sqa = jnp.sum(xall * xall, axis=0, keepdims=True)
    sqt = jnp.sum(xt * xt, axis=1, keepdims=True)
    d2 = sqt + sqa - 2.0 * jnp.dot(xt, xall, preferred_element_type=jnp.float32)
    iota = lax.broadcasted_iota(jnp.int32, (TN, N), 1)
    lane = lax.broadcasted_iota(jnp.int32, (TN, K), 1)
    acc = jnp.zeros((TN, K), jnp.int32)
    d = d2
    for k in range(K):
        m = jnp.min(d, axis=1, keepdims=True)
        eq = d == m
        ik = jnp.min(jnp.where(eq, iota, N), axis=1, keepdims=True)
        acc = jnp.where(lane == k, ik, acc)
        d = jnp.where(eq, jnp.inf, d)
    idx_ref[...] = acc + b * N


def _proj_knn(xp, xrp, yT, wlsT, wkeyT, wqT, bias):
    nt = N // TN
    f32 = jnp.float32
    return pl.pallas_call(
        _proj_knn_body,
        grid=(B, nt),
        in_specs=[
            pl.BlockSpec((1, 8, N), lambda b, i: (b, 0, 0)),
            pl.BlockSpec((TN, 8), lambda b, i: (b * nt + i, 0)),
            pl.BlockSpec((TN, D), lambda b, i: (b * nt + i, 0)),
            pl.BlockSpec((D, D), lambda b, i: (0, 0)),
            pl.BlockSpec((D, D), lambda b, i: (0, 0)),
            pl.BlockSpec((8, D), lambda b, i: (0, 0)),
            pl.BlockSpec((8, D), lambda b, i: (0, 0)),
        ],
        out_specs=[
            pl.BlockSpec((TN, D), lambda b, i: (b * nt + i, 0)),
            pl.BlockSpec((TN, 2 * D), lambda b, i: (b * nt + i, 0)),
            pl.BlockSpec((TN, D), lambda b, i: (b * nt + i, 0)),
            pl.BlockSpec((TN, K), lambda b, i: (b * nt + i, 0)),
        ],
        out_shape=[
            jax.ShapeDtypeStruct((BN, D), f32),
            jax.ShapeDtypeStruct((BN, 2 * D), f32),
            jax.ShapeDtypeStruct((BN, D), f32),
            jax.ShapeDtypeStruct((BN, K), jnp.int32),
        ],
    )(xp, xrp, yT, wlsT, wkeyT, wqT, bias)


NBUF = 4
NCH = PW // CHUNK


def _sc_gather_body(tab_hbm, idx_hbm, g_hbm, idx_v,
                    b0, b1, b2, b3, s0, s1, s2, s3):
    wid = lax.axis_index("s") * 2 + lax.axis_index("c")
    base = wid * PW
    pltpu.sync_copy(idx_hbm.at[pl.ds(base, PW)], idx_v)
    bufs = [(b0, s0), (b1, s1), (b2, s2), (b3, s3)]

    for b in range(NBUF):
        ids = idx_v.at[pl.ds(b * CHUNK, CHUNK)]
        pltpu.async_copy(tab_hbm.at[ids], bufs[b][0], bufs[b][1])

    def outer(g, carry):
        for b in range(NBUF):
            i = g * NBUF + b
            buf, sem = bufs[b]
            pltpu.make_async_copy(tab_hbm.at[pl.ds(0, CHUNK)], buf, sem).wait()
            pltpu.sync_copy(buf, g_hbm.at[pl.ds(base + i * CHUNK, CHUNK)])
            nxt = i + NBUF

            @pl.when(nxt < NCH)
            def _():
                ids2 = idx_v.at[pl.ds(nxt * CHUNK, CHUNK)]
                pltpu.async_copy(tab_hbm.at[ids2], buf, sem)
        return carry

    lax.fori_loop(0, NCH // NBUF, outer, 0)


def _sc_gather(tab_r, idx_flat):
    f32 = jnp.float32
    mesh = plsc.VectorSubcoreMesh(core_axis_name="c", subcore_axis_name="s")
    kern = pl.kernel(
        _sc_gather_body,
        mesh=mesh,
        out_type=jax.ShapeDtypeStruct((BNK, 2 * D), f32),
        scratch_types=(
            [pltpu.VMEM((PW,), jnp.int32)]
            + [pltpu.VMEM((CHUNK, 2 * D), f32)] * NBUF
            + [pltpu.SemaphoreType.DMA] * NBUF
        ),
    )
    return kern(tab_r, idx_flat)


def _accum(ref, part):
    @pl.when(pl.program_id(0) == 0)
    def _():
        ref[...] = part

    @pl.when(pl.program_id(0) != 0)
    def _():
        ref[...] += part


def _pos_stats_body(g_ref, xr_ref, sp_ref, mp_ref):
    xg = g_ref[...][:, :, D:D + 16]
    pr = xr_ref[...][:, None, :] - xg
    pr2 = pr.reshape(pr.shape[0] * K, 16)
    m = lax.dot_general(pr2, pr2, (((0,), (0,)), ((), ())),
                        preferred_element_type=jnp.float32)
    s = jnp.sum(pr2, axis=0, keepdims=True)
    _accum(mp_ref, m)
    _accum(sp_ref, jnp.broadcast_to(s, (8, 16)))


def _pos_stats(g3, x16):
    gp = 512
    nt = BN // gp
    f32 = jnp.float32
    return pl.pallas_call(
        _pos_stats_body,
        grid=(nt,),
        in_specs=[
            pl.BlockSpec((gp, K, 2 * D), lambda i: (i, 0, 0)),
            pl.BlockSpec((gp, 16), lambda i: (i, 0)),
        ],
        out_specs=[
            pl.BlockSpec((8, 16), lambda i: (0, 0)),
            pl.BlockSpec((16, 16), lambda i: (0, 0)),
        ],
        out_shape=[
            jax.ShapeDtypeStruct((8, 16), f32),
            jax.ShapeDtypeStruct((16, 16), f32),
        ],
    )(g3, x16)


def _pos_branch(xg, xr_ref, w1f_ref, wp2_ref, pk_ref):
    gr = xr_ref.shape[0]
    pr = xr_ref[...][:, None, :] - xg
    pr2 = pr.reshape(gr * K, 16)
    h = jnp.dot(pr2, w1f_ref[...], preferred_element_type=jnp.float32)
    h = jnp.maximum(h + pk_ref[0:1, :], 0.0)
    pe = jnp.dot(h, wp2_ref[...], preferred_element_type=jnp.float32)
    return pe + pk_ref[1:2, :]


def _t_stats_body(g_ref, xr_ref, q_ref, w1f_ref, wp2_ref, pk_ref,
                  st_ref, mt_ref):
    g = g_ref[...]
    pe = _pos_branch(g[:, :, D:D + 16], xr_ref, w1f_ref, wp2_ref, pk_ref)
    qk = q_ref[...][:, None, :] - g[:, :, :D]
    t = qk.reshape(GR * K, D) + pe
    m = lax.dot_general(t, t, (((0,), (0,)), ((), ())),
                        preferred_element_type=jnp.float32)
    s = jnp.sum(t, axis=0, keepdims=True)
    _accum(mt_ref, m)
    _accum(st_ref, jnp.broadcast_to(s, (8, D)))


def _t_stats(g3, x16, q_r, w1f, wp2T, pk64):
    nt = BN // GR
    f32 = jnp.float32
    return pl.pallas_call(
        _t_stats_body,
        grid=(nt,),
        in_specs=[
            pl.BlockSpec((GR, K, 2 * D), lambda i: (i, 0, 0)),
            pl.BlockSpec((GR, 16), lambda i: (i, 0)),
            pl.BlockSpec((GR, D), lambda i: (i, 0)),
            pl.BlockSpec((16, D), lambda i: (0, 0)),
            pl.BlockSpec((D, D), lambda i: (0, 0)),
            pl.BlockSpec((8, D), lambda i: (0, 0)),
        ],
        out_specs=[
            pl.BlockSpec((8, D), lambda i: (0, 0)),
            pl.BlockSpec((D, D), lambda i: (0, 0)),
        ],
        out_shape=[
            jax.ShapeDtypeStruct((8, D), f32),
            jax.ShapeDtypeStruct((D, D), f32),
        ],
    )(g3, x16, q_r, w1f, wp2T, pk64)


def _final_body(g_ref, xr_ref, q_ref, v_ref, id_ref,
                w1f_ref, wp2_ref, pk_ref, a1f_ref, pk256_ref, wa2_ref,
                wle_ref, out_ref):
    g = g_ref[...]
    pe = _pos_branch(g[:, :, D:D + 16], xr_ref, w1f_ref, wp2_ref, pk_ref)
    qk = q_ref[...][:, None, :] - g[:, :, :D]
    t = qk.reshape(GR * K, D) + pe
    a = jnp.dot(t, a1f_ref[...], preferred_element_type=jnp.float32)
    a = jnp.maximum(a + pk256_ref[0:1, :], 0.0)
    lg = jnp.dot(a, wa2_ref[...], preferred_element_type=jnp.float32)
    lg = lg + pk_ref[2:3, :]
    lg3 = lg.reshape(GR, K, D)
    mx = jnp.max(lg3, axis=1, keepdims=True)
    e = jnp.exp(lg3 - mx)
    attn = e / jnp.sum(e, axis=1, keepdims=True)
    val = v_ref[...][:, None, :] + pe.reshape(GR, K, D)
    agg = jnp.sum(attn * val, axis=1)
    out = jnp.dot(agg, wle_ref[...], preferred_element_type=jnp.float32)
    out_ref[...] = out + pk_ref[3:4, :] + id_ref[...]


def _final(g3, x16, q_r, y1_r, yT, w1f, wp2T, pk64, a1f, pk256, wa2T, wleT):
    nt = BN // GR
    f32 = jnp.float32
    return pl.pallas_call(
        _final_body,
        grid=(nt,),
        in_specs=[
            pl.BlockSpec((GR, K, 2 * D), lambda i: (i, 0, 0)),
            pl.BlockSpec((GR, 16), lambda i: (i, 0)),
            pl.BlockSpec((GR, D), lambda i: (i, 0)),
            pl.BlockSpec((GR, D), lambda i: (i, 0)),
            pl.BlockSpec((GR, D), lambda i: (i, 0)),
            pl.BlockSpec((16, D), lambda i: (0, 0)),
            pl.BlockSpec((D, D), lambda i: (0, 0)),
            pl.BlockSpec((8, D), lambda i: (0, 0)),
            pl.BlockSpec((D, 4 * D), lambda i: (0, 0)),
            pl.BlockSpec((8, 4 * D), lambda i: (0, 0)),
            pl.BlockSpec((4 * D, D), lambda i: (0, 0)),
            pl.BlockSpec((D, D), lambda i: (0, 0)),
        ],
        out_specs=[pl.BlockSpec((GR, D), lambda i: (i, 0))],
        out_shape=[jax.ShapeDtypeStruct((BN, D), f32)],
    )(g3, x16, q_r, y1_r, yT, w1f, wp2T, pk64, a1f, pk256, wa2T, wleT)[0]


def kernel(x, y, W_ls, b_ls, W_key, b_key, W_q, b_q, W_p1, b_p1, g_p, be_p,
           W_p2, b_p2, W_a1, b_a1, g_a, be_a, W_a2, b_a2, W_le, b_le):
    f32 = jnp.float32
    xp = jnp.pad(x, ((0, 0), (0, 5), (0, 0)))
    xr = jnp.transpose(x, (0, 2, 1))
    xrp = jnp.pad(xr, ((0, 0), (0, 0), (0, 5))).reshape(BN, 8)
    x16 = jnp.pad(xr, ((0, 0), (0, 0), (0, 13))).reshape(BN, 16)
    yT = jnp.transpose(y, (0, 2, 1)).reshape(BN, D)

    W_qp = jnp.pad(W_q, ((0, 0), (0, 5)))
    bias = jnp.zeros((8, D), f32).at[0].set(b_ls).at[1].set(b_key).at[2].set(b_q)

    y1_r, tab_r, q_r, idxg = _proj_knn(
        xp, xrp, yT, W_ls.T, W_key.T, W_qp.T, bias)

    g3 = _sc_gather(tab_r, idxg.reshape(BNK)).reshape(BN, K, 2 * D)

    sp, mp = _pos_stats(g3, x16)
    cnt = float(BNK)
    mu_p = sp[0] / cnt
    cov_p = mp / cnt - jnp.outer(mu_p, mu_p)
    W1p = jnp.pad(W_p1, ((0, 0), (0, 13)))
    m_o = W1p @ mu_p + b_p1
    v_o = ((W1p @ cov_p) * W1p).sum(1)
    s_p = g_p / jnp.sqrt(v_o + EPS)
    c_p = s_p * (b_p1 - m_o) + be_p
    w1f = W1p.T * s_p[None, :]
    pk64 = jnp.zeros((8, D), f32).at[0].set(c_p).at[1].set(b_p2)

    st, mt = _t_stats(g3, x16, q_r, w1f, W_p2.T, pk64)
    mu_t = st[0] / cnt
    cov_t = mt / cnt - jnp.outer(mu_t, mu_t)
    m_a = W_a1 @ mu_t + b_a1
    v_a = ((W_a1 @ cov_t) * W_a1).sum(1)
    s_a = g_a / jnp.sqrt(v_a + EPS)
    c_a = s_a * (b_a1 - m_a) + be_a
    a1f = W_a1.T * s_a[None, :]
    pk64 = pk64.at[2].set(b_a2).at[3].set(b_le)
    pk256 = jnp.zeros((8, 4 * D), f32).at[0].set(c_a)

    out_r = _final(g3, x16, q_r, y1_r, yT,
                   w1f, W_p2.T, pk64, a1f, pk256, W_a2.T, W_le.T)
    return jnp.transpose(out_r.reshape(B, N, D), (0, 2, 1))


# speedup vs baseline: 16.7469x; 1.0139x over previous
import functools

import jax
import jax.numpy as jnp
from jax import lax
from jax.experimental import pallas as pl
from jax.experimental.pallas import tpu as pltpu
from jax.experimental.pallas import tpu_sc as plsc

D = 64
K = 16
B = 4
N = 4096
BN = B * N
BNK = BN * K
EPS = 1e-5

TN = 256
GR = 512
NW = 32
CHUNK = 128
PW = BNK // NW


def _proj_knn_body(xall_ref, xt_ref, yt_ref, wls_ref, wkey_ref, wq_ref,
                   bias_ref, y1_ref, tab_ref, q_ref, idx_ref):
    b = pl.program_id(0)
    xall = xall_ref[0]
    xt = xt_ref[...]
    yt = yt_ref[...]
    y1 = jnp.dot(yt, wls_ref[...], preferred_element_type=jnp.float32)
    y1 = y1 + bias_ref[0:1, :]
    key = jnp.dot(y1, wkey_ref[...], preferred_element_type=jnp.float32)
    key = key + bias_ref[1:2, :]
    q = jnp.dot(xt, wq_ref[...], preferred_element_type=jnp.float32)
    q = q + bias_ref[2:3, :]
    y1_ref[...] = y1
    tab_ref[...] = jnp.concatenate(
        [key, jnp.pad(xt, ((0, 0), (0, 56)))], axis=1)
    q_ref[...] = q

    sqa = jnp.sum(xall * xall, axis=0, keepdims=True)
    sqt = jnp.sum(xt * xt, axis=1, keepdims=True)
    d2 = sqt + sqa - 2.0 * jnp.dot(xt, xall, preferred_element_type=jnp.float32)
    iota = lax.broadcasted_iota(jnp.int32, (TN, N), 1)
    lane = lax.broadcasted_iota(jnp.int32, (TN, K), 1)
    acc = jnp.zeros((TN, K), jnp.int32)
    d = d2
    for k in range(K):
        m = jnp.min(d, axis=1, keepdims=True)
        eq = d == m
        ik = jnp.min(jnp.where(eq, iota, N), axis=1, keepdims=True)
        acc = jnp.where(lane == k, ik, acc)
        d = jnp.where(eq, jnp.inf, d)
    idx_ref[...] = acc + b * N


def _proj_knn(xp, xrp, yT, wlsT, wkeyT, wqT, bias):
    nt = N // TN
    f32 = jnp.float32
    return pl.pallas_call(
        _proj_knn_body,
        grid=(B, nt),
        in_specs=[
            pl.BlockSpec((1, 8, N), lambda b, i: (b, 0, 0)),
            pl.BlockSpec((TN, 8), lambda b, i: (b * nt + i, 0)),
            pl.BlockSpec((TN, D), lambda b, i: (b * nt + i, 0)),
            pl.BlockSpec((D, D), lambda b, i: (0, 0)),
            pl.BlockSpec((D, D), lambda b, i: (0, 0)),
            pl.BlockSpec((8, D), lambda b, i: (0, 0)),
            pl.BlockSpec((8, D), lambda b, i: (0, 0)),
        ],
        out_specs=[
            pl.BlockSpec((TN, D), lambda b, i: (b * nt + i, 0)),
            pl.BlockSpec((TN, 2 * D), lambda b, i: (b * nt + i, 0)),
            pl.BlockSpec((TN, D), lambda b, i: (b * nt + i, 0)),
            pl.BlockSpec((TN, K), lambda b, i: (b * nt + i, 0)),
        ],
        out_shape=[
            jax.ShapeDtypeStruct((BN, D), f32),
            jax.ShapeDtypeStruct((BN, 2 * D), f32),
            jax.ShapeDtypeStruct((BN, D), f32),
            jax.ShapeDtypeStruct((BN, K), jnp.int32),
        ],
    )(xp, xrp, yT, wlsT, wkeyT, wqT, bias)


NBUF = 4
NCH = PW // CHUNK


def _sc_gather_body(tab_hbm, idx_hbm, g_hbm, idx_v,
                    b0, b1, b2, b3, s0, s1, s2, s3):
    wid = lax.axis_index("s") * 2 + lax.axis_index("c")
    base = wid * PW
    pltpu.sync_copy(idx_hbm.at[pl.ds(base, PW)], idx_v)
    bufs = [(b0, s0), (b1, s1), (b2, s2), (b3, s3)]

    for b in range(NBUF):
        ids = idx_v.at[pl.ds(b * CHUNK, CHUNK)]
        pltpu.async_copy(tab_hbm.at[ids], bufs[b][0], bufs[b][1])

    def outer(g, carry):
        for b in range(NBUF):
            i = g * NBUF + b
            buf, sem = bufs[b]
            pltpu.make_async_copy(tab_hbm.at[pl.ds(0, CHUNK)], buf, sem).wait()
            pltpu.sync_copy(buf, g_hbm.at[pl.ds(base + i * CHUNK, CHUNK)])
            nxt = i + NBUF

            @pl.when(nxt < NCH)
            def _():
                ids2 = idx_v.at[pl.ds(nxt * CHUNK, CHUNK)]
                pltpu.async_copy(tab_hbm.at[ids2], buf, sem)
        return carry

    lax.fori_loop(0, NCH // NBUF, outer, 0)


def _sc_gather(tab_r, idx_flat):
    f32 = jnp.float32
    mesh = plsc.VectorSubcoreMesh(core_axis_name="c", subcore_axis_name="s")
    kern = pl.kernel(
        _sc_gather_body,
        mesh=mesh,
        out_type=jax.ShapeDtypeStruct((BNK, 2 * D), f32),
        scratch_types=(
            [pltpu.VMEM((PW,), jnp.int32)]
            + [pltpu.VMEM((CHUNK, 2 * D), f32)] * NBUF
            + [pltpu.SemaphoreType.DMA] * NBUF
        ),
    )
    return kern(tab_r, idx_flat)


def _accum(ref, part):
    @pl.when(pl.program_id(0) == 0)
    def _():
        ref[...] = part

    @pl.when(pl.program_id(0) != 0)
    def _():
        ref[...] += part


def _pos_stats_body(g_ref, xr_ref, sp_ref, mp_ref):
    xg = g_ref[...][:, :, D:D + 16]
    pr = xr_ref[...][:, None, :] - xg
    pr2 = pr.reshape(pr.shape[0] * K, 16)
    m = lax.dot_general(pr2, pr2, (((0,), (0,)), ((), ())),
                        preferred_element_type=jnp.float32)
    s = jnp.sum(pr2, axis=0, keepdims=True)
    _accum(mp_ref, m)
    _accum(sp_ref, jnp.broadcast_to(s, (8, 16)))


def _pos_stats(g3, x16):
    gp = 512
    nt = BN // gp
    f32 = jnp.float32
    return pl.pallas_call(
        _pos_stats_body,
        grid=(nt,),
        in_specs=[
            pl.BlockSpec((gp, K, 2 * D), lambda i: (i, 0, 0)),
            pl.BlockSpec((gp, 16), lambda i: (i, 0)),
        ],
        out_specs=[
            pl.BlockSpec((8, 16), lambda i: (0, 0)),
            pl.BlockSpec((16, 16), lambda i: (0, 0)),
        ],
        out_shape=[
            jax.ShapeDtypeStruct((8, 16), f32),
            jax.ShapeDtypeStruct((16, 16), f32),
        ],
    )(g3, x16)


def _pos_branch(xg, xr_ref, w1f_ref, wp2_ref, pk_ref):
    gr = xr_ref.shape[0]
    pr = xr_ref[...][:, None, :] - xg
    pr2 = pr.reshape(gr * K, 16)
    h = jnp.dot(pr2, w1f_ref[...], preferred_element_type=jnp.float32)
    h = jnp.maximum(h + pk_ref[0:1, :], 0.0)
    pe = jnp.dot(h, wp2_ref[...], preferred_element_type=jnp.float32)
    return pe + pk_ref[1:2, :]


def _t_stats_body(g_ref, xr_ref, q_ref, w1f_ref, wp2_ref, pk_ref,
                  st_ref, mt_ref):
    g = g_ref[...]
    pe = _pos_branch(g[:, :, D:D + 16], xr_ref, w1f_ref, wp2_ref, pk_ref)
    qk = q_ref[...][:, None, :] - g[:, :, :D]
    t = qk.reshape(GR * K, D) + pe
    m = lax.dot_general(t, t, (((0,), (0,)), ((), ())),
                        preferred_element_type=jnp.float32)
    s = jnp.sum(t, axis=0, keepdims=True)
    _accum(mt_ref, m)
    _accum(st_ref, jnp.broadcast_to(s, (8, D)))


def _t_stats(g3, x16, q_r, w1f, wp2T, pk64):
    nt = BN // GR
    f32 = jnp.float32
    return pl.pallas_call(
        _t_stats_body,
        grid=(nt,),
        in_specs=[
            pl.BlockSpec((GR, K, 2 * D), lambda i: (i, 0, 0)),
            pl.BlockSpec((GR, 16), lambda i: (i, 0)),
            pl.BlockSpec((GR, D), lambda i: (i, 0)),
            pl.BlockSpec((16, D), lambda i: (0, 0)),
            pl.BlockSpec((D, D), lambda i: (0, 0)),
            pl.BlockSpec((8, D), lambda i: (0, 0)),
        ],
        out_specs=[
            pl.BlockSpec((8, D), lambda i: (0, 0)),
            pl.BlockSpec((D, D), lambda i: (0, 0)),
        ],
        out_shape=[
            jax.ShapeDtypeStruct((8, D), f32),
            jax.ShapeDtypeStruct((D, D), f32),
        ],
    )(g3, x16, q_r, w1f, wp2T, pk64)


def _final_body(g_ref, xr_ref, q_ref, v_ref, id_ref,
                w1f_ref, wp2_ref, pk_ref, a1f_ref, pk256_ref, wa2_ref,
                wle_ref, out_ref):
    g = g_ref[...]
    pe = _pos_branch(g[:, :, D:D + 16], xr_ref, w1f_ref, wp2_ref, pk_ref)
    qk = q_ref[...][:, None, :] - g[:, :, :D]
    t = qk.reshape(GR * K, D) + pe
    a = jnp.dot(t, a1f_ref[...], preferred_element_type=jnp.float32)
    a = jnp.maximum(a + pk256_ref[0:1, :], 0.0)
    lg = jnp.dot(a, wa2_ref[...], preferred_element_type=jnp.float32)
    lg = lg + pk_ref[2:3, :]
    lg3 = lg.reshape(GR, K, D)
    mx = jnp.max(lg3, axis=1, keepdims=True)
    e = jnp.exp(lg3 - mx)
    attn = e / jnp.sum(e, axis=1, keepdims=True)
    val = v_ref[...][:, None, :] + pe.reshape(GR, K, D)
    agg = jnp.sum(attn * val, axis=1)
    out = jnp.dot(agg, wle_ref[...], preferred_element_type=jnp.float32)
    out_ref[...] = out + pk_ref[3:4, :] + id_ref[...]


def _final(g3, x16, q_r, y1_r, yT, w1f, wp2T, pk64, a1f, pk256, wa2T, wleT):
    nt = BN // GR
    f32 = jnp.float32
    return pl.pallas_call(
        _final_body,
        grid=(nt,),
        in_specs=[
            pl.BlockSpec((GR, K, 2 * D), lambda i: (i, 0, 0)),
            pl.BlockSpec((GR, 16), lambda i: (i, 0)),
            pl.BlockSpec((GR, D), lambda i: (i, 0)),
            pl.BlockSpec((GR, D), lambda i: (i, 0)),
            pl.BlockSpec((GR, D), lambda i: (i, 0)),
            pl.BlockSpec((16, D), lambda i: (0, 0)),
            pl.BlockSpec((D, D), lambda i: (0, 0)),
            pl.BlockSpec((8, D), lambda i: (0, 0)),
            pl.BlockSpec((D, 4 * D), lambda i: (0, 0)),
            pl.BlockSpec((8, 4 * D), lambda i: (0, 0)),
            pl.BlockSpec((4 * D, D), lambda i: (0, 0)),
            pl.BlockSpec((D, D), lambda i: (0, 0)),
        ],
        out_specs=[pl.BlockSpec((GR, D), lambda i: (i, 0))],
        out_shape=[jax.ShapeDtypeStruct((BN, D), f32)],
    )(g3, x16, q_r, y1_r, yT, w1f, wp2T, pk64, a1f, pk256, wa2T, wleT)[0]


def kernel(x, y, W_ls, b_ls, W_key, b_key, W_q, b_q, W_p1, b_p1, g_p, be_p,
           W_p2, b_p2, W_a1, b_a1, g_a, be_a, W_a2, b_a2, W_le, b_le):
    f32 = jnp.float32
    xp = jnp.pad(x, ((0, 0), (0, 5), (0, 0)))
    xr = jnp.transpose(x, (0, 2, 1))
    xrp = jnp.pad(xr, ((0, 0), (0, 0), (0, 5))).reshape(BN, 8)
    x16 = jnp.pad(xr, ((0, 0), (0, 0), (0, 13))).reshape(BN, 16)
    yT = jnp.transpose(y, (0, 2, 1)).reshape(BN, D)

    W_qp = jnp.pad(W_q, ((0, 0), (0, 5)))
    bias = jnp.zeros((8, D), f32).at[0].set(b_ls).at[1].set(b_key).at[2].set(b_q)

    y1_r, tab_r, q_r, idxg = _proj_knn(
        xp, xrp, yT, W_ls.T, W_key.T, W_qp.T, bias)

    g3 = _sc_gather(tab_r, idxg.reshape(BNK)).reshape(BN, K, 2 * D)

    sp, mp = _pos_stats(g3, x16)
    cnt = float(BNK)
    mu_p = sp[0] / cnt
    cov_p = mp / cnt - jnp.outer(mu_p, mu_p)
    W1p = jnp.pad(W_p1, ((0, 0), (0, 13)))
    m_o = W1p @ mu_p + b_p1
    v_o = ((W1p @ cov_p) * W1p).sum(1)
    s_p = g_p / jnp.sqrt(v_o + EPS)
    c_p = s_p * (b_p1 - m_o) + be_p
    w1f = W1p.T * s_p[None, :]
    pk64 = jnp.zeros((8, D), f32).at[0].set(c_p).at[1].set(b_p2)

    st, mt = _t_stats(g3, x16, q_r, w1f, W_p2.T, pk64)
    mu_t = st[0] / cnt
    cov_t = mt / cnt - jnp.outer(mu_t, mu_t)
    m_a = W_a1 @ mu_t + b_a1
    v_a = ((W_a1 @ cov_t) * W_a1).sum(1)
    s_a = g_a / jnp.sqrt(v_a + EPS)
    c_a = s_a * (b_a1 - m_a) + be_a
    a1f = W_a1.T * s_a[None, :]
    pk64 = pk64.at[2].set(b_a2).at[3].set(b_le)
    pk256 = jnp.zeros((8, 4 * D), f32).at[0].set(c_a)

    out_r = _final(g3, x16, q_r, y1_r, yT,
                   w1f, W_p2.T, pk64, a1f, pk256, W_a2.T, W_le.T)
    return jnp.transpose(out_r.reshape(B, N, D), (0, 2, 1))
